# Initial kernel scaffold; baseline (speedup 1.0000x reference)
import jax
import jax.numpy as jnp
from jax.experimental import pallas as pl


def kernel(x, edge_attr, params, edge_index, batch):
    raise NotImplementedError("write your pallas kernel here")


# speedup vs baseline: 1.2509x; 1.2509x over previous
import functools

import jax
import jax.numpy as jnp
from jax import lax
from jax.experimental import pallas as pl
from jax.experimental.pallas import tpu as pltpu
from jax.experimental.pallas import tpu_sc as plsc

_N = 10000
_E = 160000
_D = 256
_G = 64
_BN_EPS = 1e-5

_NT = 32
_RPT = 312
_LAST_EXTRA = 16
_BUF_ROWS = _RPT + _LAST_EXTRA + 1
_DUMMY = _RPT + _LAST_EXTRA
_C = 40
_EPAD = _E + 2 * _C


def _read_scalar_static(vref, idx, nchunks):
    lanes = lax.iota(jnp.int32, 16)
    acc = jnp.int32(0)
    for c in range(nchunks):
        v = vref[pl.ds(c * 16, 16)]
        acc = acc + jnp.sum(jnp.where(lanes + (c * 16) == idx, v, 0))
    return acc


def _read_scalar_dyn(vref, idx):
    lanes = lax.iota(jnp.int32, 16)
    base = (idx // 16) * 16
    v = vref[pl.ds(base, 16)]
    return jnp.sum(jnp.where(lanes == idx - base, v, 0))


def _sc_message_kernel(x, ea, srcs, dsts, perm, bounds):
    n, d = x.shape
    mesh = plsc.VectorSubcoreMesh(core_axis_name="c", subcore_axis_name="s")
    nj = d // 16

    @functools.partial(
        pl.kernel,
        out_type=jax.ShapeDtypeStruct((n * d,), jnp.float32),
        mesh=mesh,
        compiler_params=pltpu.CompilerParams(needs_layout_passes=False),
        scratch_types=[
            pltpu.VMEM((_BUF_ROWS * _D,), jnp.float32),
            pltpu.VMEM((_C, _D), jnp.float32),
            pltpu.VMEM((_C, _D), jnp.float32),
            pltpu.VMEM((_C,), jnp.int32),
            pltpu.VMEM((48,), jnp.int32),
            pltpu.VMEM((_C,), jnp.int32),
            pltpu.VMEM((48,), jnp.int32),
            pltpu.SemaphoreType.DMA,
            pltpu.SemaphoreType.DMA,
        ],
    )
    def k(x_hbm, ea_hbm, src_hbm, dst_hbm, perm_hbm, bnd_hbm, out_hbm,
          aggbuf, xbuf, eabuf, sidx, didx, pidx, bbuf, sem1, sem2):
        nc = 2
        wid = lax.axis_index("s") * nc + lax.axis_index("c")
        nbase = wid * _RPT

        def zero_body(i, _):
            aggbuf[pl.ds(i * 16, 16)] = jnp.zeros((16,), jnp.float32)
            return 0
        lax.fori_loop(0, _BUF_ROWS * nj, zero_body, 0)

        pltpu.sync_copy(bnd_hbm, bbuf)
        b_lo = _read_scalar_static(bbuf, wid, 3)
        b_hi = _read_scalar_static(bbuf, wid + 1, 3)
        lo_al = (b_lo // 8) * 8
        nch = (b_hi - lo_al + (_C - 1)) // _C

        def chunk_body(i, _):
            k0 = lo_al + i * _C
            pltpu.sync_copy(src_hbm.at[pl.ds(k0, _C)], sidx)
            pltpu.sync_copy(dst_hbm.at[pl.ds(k0, _C)], didx.at[pl.ds(0, _C)])
            pltpu.sync_copy(perm_hbm.at[pl.ds(k0, _C)], pidx)
            cp1 = pltpu.async_copy(x_hbm.at[sidx], xbuf, sem1)
            cp2 = pltpu.async_copy(ea_hbm.at[pidx], eabuf, sem2)
            cp1.wait()
            cp2.wait()

            def edge_body(e, _):
                ge = k0 + e
                valid = (ge >= b_lo) & (ge < b_hi)
                drow = _read_scalar_dyn(didx, e)
                rowloc = jnp.where(valid, drow - nbase, _DUMMY)
                boff = rowloc * _D
                for j in range(nj):
                    xv = xbuf[e, pl.ds(j * 16, 16)]
                    ev = eabuf[e, pl.ds(j * 16, 16)]
                    m = jnp.maximum(xv + ev, 0.0)
                    plsc.addupdate(aggbuf.at[pl.ds(boff + j * 16, 16)], m)
                return 0
            lax.fori_loop(0, _C, edge_body, 0)
            return 0
        lax.fori_loop(0, nch, chunk_body, 0)

        pltpu.sync_copy(aggbuf.at[pl.ds(0, _RPT * _D)],
                        out_hbm.at[pl.ds(nbase * _D, _RPT * _D)])

        @pl.when(wid == _NT - 1)
        def _():
            pltpu.sync_copy(aggbuf.at[pl.ds(_RPT * _D, _LAST_EXTRA * _D)],
                            out_hbm.at[pl.ds((_NT * _RPT) * _D,
                                             _LAST_EXTRA * _D)])

    return k(x, ea, srcs, dsts, perm, bounds)


_BN_ROWS = 2000


def _pool_dot(pt_block, h_block):
    return lax.dot_general(pt_block, h_block, (((0,), (0,)), ((), ())),
                           preferred_element_type=jnp.float32)


def _layer_tc_kernel(agg, xcur, w1f, b1f, w2f, b2f, pmat_t):
    n, d = xcur.shape
    g = pmat_t.shape[1]
    grid = n // _BN_ROWS

    def body(agg_ref, x_ref, w1_ref, b1_ref, w2_ref, b2_ref, p_ref,
             h_ref, pool_ref):
        z = agg_ref[...] + x_ref[...]
        t = jnp.dot(z, w1_ref[...], preferred_element_type=jnp.float32)
        t = jnp.maximum(t + b1_ref[...], 0.0)
        h = jnp.dot(t, w2_ref[...], preferred_element_type=jnp.float32)
        h = jnp.maximum(h + b2_ref[...], 0.0)
        h_ref[...] = h

        @pl.when(pl.program_id(0) == 0)
        def _():
            pool_ref[...] = jnp.zeros_like(pool_ref)
        pool_ref[...] += _pool_dot(p_ref[...], h)

    return pl.pallas_call(
        body,
        grid=(grid,),
        in_specs=[
            pl.BlockSpec((_BN_ROWS, d), lambda i: (i, 0)),
            pl.BlockSpec((_BN_ROWS, d), lambda i: (i, 0)),
            pl.BlockSpec((d, d), lambda i: (0, 0)),
            pl.BlockSpec((1, d), lambda i: (0, 0)),
            pl.BlockSpec((d, d), lambda i: (0, 0)),
            pl.BlockSpec((1, d), lambda i: (0, 0)),
            pl.BlockSpec((_BN_ROWS, g), lambda i: (i, 0)),
        ],
        out_specs=[
            pl.BlockSpec((_BN_ROWS, d), lambda i: (i, 0)),
            pl.BlockSpec((g, d), lambda i: (0, 0)),
        ],
        out_shape=[
            jax.ShapeDtypeStruct((n, d), jnp.float32),
            jax.ShapeDtypeStruct((g, d), jnp.float32),
        ],
    )(agg, xcur, w1f, b1f, w2f, b2f, pmat_t)


def _pool_tc_kernel(xcur, pmat_t):
    n, d = xcur.shape
    g = pmat_t.shape[1]
    grid = n // _BN_ROWS

    def body(x_ref, p_ref, pool_ref):
        @pl.when(pl.program_id(0) == 0)
        def _():
            pool_ref[...] = jnp.zeros_like(pool_ref)
        pool_ref[...] += _pool_dot(p_ref[...], x_ref[...])

    return pl.pallas_call(
        body,
        grid=(grid,),
        in_specs=[
            pl.BlockSpec((_BN_ROWS, d), lambda i: (i, 0)),
            pl.BlockSpec((_BN_ROWS, g), lambda i: (i, 0)),
        ],
        out_specs=pl.BlockSpec((g, d), lambda i: (0, 0)),
        out_shape=jax.ShapeDtypeStruct((g, d), jnp.float32),
    )(xcur, pmat_t)


def _final_tc_kernel(pooled_cat, wcat, bsum):
    g, dk = pooled_cat.shape
    dout = wcat.shape[1]

    def body(p_ref, w_ref, b_ref, o_ref):
        o_ref[...] = jnp.dot(p_ref[...], w_ref[...],
                             preferred_element_type=jnp.float32) + b_ref[...]

    return pl.pallas_call(
        body,
        out_shape=jax.ShapeDtypeStruct((g, dout), jnp.float32),
    )(pooled_cat, wcat, bsum)


def kernel(x, edge_attr, params, edge_index, batch):
    n, d = x.shape
    e = edge_index.shape[1]
    g = _G

    src = edge_index[0]
    dst = edge_index[1]
    perm = jnp.argsort(dst)
    dsts = dst[perm]
    srcs = src[perm]

    node_starts = jnp.concatenate(
        [jnp.arange(_NT, dtype=jnp.int32) * _RPT,
         jnp.array([n], dtype=jnp.int32)])
    bounds = jnp.searchsorted(dsts, node_starts, side="left").astype(jnp.int32)
    bounds = jnp.concatenate(
        [bounds, jnp.full((48 - _NT - 1,), e, dtype=jnp.int32)])

    pad = _EPAD - e
    srcs_p = jnp.concatenate([srcs, jnp.zeros((pad,), jnp.int32)])
    dsts_p = jnp.concatenate([dsts, jnp.zeros((pad,), jnp.int32)])
    perm_p = jnp.concatenate([perm.astype(jnp.int32),
                              jnp.zeros((pad,), jnp.int32)])

    onehot = (batch[:, None] == jnp.arange(g, dtype=batch.dtype)[None, :])
    onehot = onehot.astype(jnp.float32)
    counts = jnp.clip(jnp.sum(onehot, axis=0, keepdims=True), 1.0)
    pmat_t = onehot / counts

    inv = 1.0 / jnp.sqrt(1.0 + _BN_EPS)
    folded = []
    for l in range(5):
        s1 = params[f'conv{l}_bn_g'] * inv
        w1f = params[f'conv{l}_W1'] * s1[None, :]
        b1f = (params[f'conv{l}_b1'] * s1 + params[f'conv{l}_bn_b'])
        s2 = params[f'bn{l}_g'] * inv
        w2f = params[f'conv{l}_W2'] * s2[None, :]
        b2f = (params[f'conv{l}_b2'] * s2 + params[f'bn{l}_b'])
        folded.append((w1f, b1f.reshape(1, -1), w2f, b2f.reshape(1, -1)))

    wcat = jnp.concatenate([params[f'fc{i}_W'] for i in range(6)], axis=0)
    bsum = sum(params[f'fc{i}_b'] for i in range(6)).reshape(1, -1)

    h = x
    pooled_list = [_pool_tc_kernel(x, pmat_t)]
    for l in range(5):
        aggflat = _sc_message_kernel(h, edge_attr, srcs_p, dsts_p, perm_p,
                                     bounds)
        agg = aggflat.reshape(n, d)
        w1f, b1f, w2f, b2f = folded[l]
        h, pooled = _layer_tc_kernel(agg, h, w1f, b1f, w2f, b2f, pmat_t)
        pooled_list.append(pooled)

    pooled_cat = jnp.concatenate(pooled_list, axis=1)
    return _final_tc_kernel(pooled_cat, wcat, bsum)


# speedup vs baseline: 2.1663x; 1.7318x over previous
import functools

import jax
import jax.numpy as jnp
from jax import lax
from jax.experimental import pallas as pl
from jax.experimental.pallas import tpu as pltpu
from jax.experimental.pallas import tpu_sc as plsc

_N = 10000
_E = 160000
_D = 256
_G = 64
_BN_EPS = 1e-5

_NT = 32
_RPT = 312
_LAST_EXTRA = 16
_BUF_ROWS = _RPT + _LAST_EXTRA + 1
_DUMMY = _RPT + _LAST_EXTRA
_C = 40
_EPAD = _E + 2 * _C


def _read_scalar_static(vref, idx, nchunks):
    lanes = lax.iota(jnp.int32, 16)
    acc = jnp.int32(0)
    for c in range(nchunks):
        v = vref[pl.ds(c * 16, 16)]
        acc = acc + jnp.sum(jnp.where(lanes + (c * 16) == idx, v, 0))
    return acc


def _read_scalar_dyn(vref, idx):
    lanes = lax.iota(jnp.int32, 16)
    base = (idx // 16) * 16
    v = vref[pl.ds(base, 16)]
    return jnp.sum(jnp.where(lanes == idx - base, v, 0))


def _sc_message_kernel(x, ea, srcs, dsts, perm, bounds):
    n, d = x.shape
    mesh = plsc.VectorSubcoreMesh(core_axis_name="c", subcore_axis_name="s")
    nj = d // 16

    @functools.partial(
        pl.kernel,
        out_type=jax.ShapeDtypeStruct((n * d,), jnp.float32),
        mesh=mesh,
        compiler_params=pltpu.CompilerParams(needs_layout_passes=False),
        scratch_types=[
            pltpu.VMEM((_BUF_ROWS * _D,), jnp.float32),
            pltpu.VMEM((_C, _D), jnp.float32),
            pltpu.VMEM((_C, _D), jnp.float32),
            pltpu.VMEM((_C,), jnp.int32),
            pltpu.VMEM((48,), jnp.int32),
            pltpu.VMEM((_C,), jnp.int32),
            pltpu.VMEM((48,), jnp.int32),
            pltpu.SemaphoreType.DMA,
            pltpu.SemaphoreType.DMA,
        ],
    )
    def k(x_hbm, ea_hbm, src_hbm, dst_hbm, perm_hbm, bnd_hbm, out_hbm,
          aggbuf, xbuf, eabuf, sidx, didx, pidx, bbuf, sem1, sem2):
        nc = 2
        wid = lax.axis_index("s") * nc + lax.axis_index("c")
        nbase = wid * _RPT
        lanes = lax.iota(jnp.int32, 16)

        @plsc.parallel_loop(0, _BUF_ROWS * nj, 1, unroll=8)
        def zero_body(i):
            aggbuf[pl.ds(i * 16, 16)] = jnp.zeros((16,), jnp.float32)

        pltpu.sync_copy(bnd_hbm, bbuf)
        b_lo = _read_scalar_static(bbuf, wid, 3)
        b_hi = _read_scalar_static(bbuf, wid + 1, 3)
        lo_al = (b_lo // 8) * 8
        nch = (b_hi - lo_al + (_C - 1)) // _C
        b_lo_v = jnp.full((16,), 0, jnp.int32) + b_lo
        b_hi_v = jnp.full((16,), 0, jnp.int32) + b_hi

        def chunk_body(i, _):
            k0 = lo_al + i * _C
            pltpu.sync_copy(src_hbm.at[pl.ds(k0, _C)], sidx)
            pltpu.sync_copy(dst_hbm.at[pl.ds(k0, _C)], didx.at[pl.ds(0, _C)])
            pltpu.sync_copy(perm_hbm.at[pl.ds(k0, _C)], pidx)
            cp1 = pltpu.async_copy(x_hbm.at[sidx], xbuf, sem1)
            cp2 = pltpu.async_copy(ea_hbm.at[pidx], eabuf, sem2)
            cp1.wait()
            cp2.wait()

            @plsc.parallel_loop(0, _C, 1, unroll=4)
            def edge_body(e):
                ge_v = (jnp.full((16,), 0, jnp.int32) + k0) + e
                e_v = jnp.full((16,), 0, jnp.int32) + e
                drow_v = plsc.load_gather(didx, [e_v])
                valid_v = (ge_v >= b_lo_v) & (ge_v < b_hi_v)
                rowloc_v = jnp.where(valid_v, drow_v - nbase, _DUMMY)
                addr0_v = rowloc_v * _D + lanes
                for j in range(nj):
                    xv = xbuf[e, pl.ds(j * 16, 16)]
                    ev = eabuf[e, pl.ds(j * 16, 16)]
                    m = jnp.maximum(xv + ev, 0.0)
                    plsc.addupdate_scatter(aggbuf, [addr0_v + j * 16], m)
            return 0
        lax.fori_loop(0, nch, chunk_body, 0)

        pltpu.sync_copy(aggbuf.at[pl.ds(0, _RPT * _D)],
                        out_hbm.at[pl.ds(nbase * _D, _RPT * _D)])

        @pl.when(wid == _NT - 1)
        def _():
            pltpu.sync_copy(aggbuf.at[pl.ds(_RPT * _D, _LAST_EXTRA * _D)],
                            out_hbm.at[pl.ds((_NT * _RPT) * _D,
                                             _LAST_EXTRA * _D)])

    return k(x, ea, srcs, dsts, perm, bounds)


_BN_ROWS = 2000


def _pool_dot(pt_block, h_block):
    return lax.dot_general(pt_block, h_block, (((0,), (0,)), ((), ())),
                           preferred_element_type=jnp.float32)


def _layer_tc_kernel(agg, xcur, w1f, b1f, w2f, b2f, pmat_t):
    n, d = xcur.shape
    g = pmat_t.shape[1]
    grid = n // _BN_ROWS

    def body(agg_ref, x_ref, w1_ref, b1_ref, w2_ref, b2_ref, p_ref,
             h_ref, pool_ref):
        z = agg_ref[...] + x_ref[...]
        t = jnp.dot(z, w1_ref[...], preferred_element_type=jnp.float32)
        t = jnp.maximum(t + b1_ref[...], 0.0)
        h = jnp.dot(t, w2_ref[...], preferred_element_type=jnp.float32)
        h = jnp.maximum(h + b2_ref[...], 0.0)
        h_ref[...] = h

        @pl.when(pl.program_id(0) == 0)
        def _():
            pool_ref[...] = jnp.zeros_like(pool_ref)
        pool_ref[...] += _pool_dot(p_ref[...], h)

    return pl.pallas_call(
        body,
        grid=(grid,),
        in_specs=[
            pl.BlockSpec((_BN_ROWS, d), lambda i: (i, 0)),
            pl.BlockSpec((_BN_ROWS, d), lambda i: (i, 0)),
            pl.BlockSpec((d, d), lambda i: (0, 0)),
            pl.BlockSpec((1, d), lambda i: (0, 0)),
            pl.BlockSpec((d, d), lambda i: (0, 0)),
            pl.BlockSpec((1, d), lambda i: (0, 0)),
            pl.BlockSpec((_BN_ROWS, g), lambda i: (i, 0)),
        ],
        out_specs=[
            pl.BlockSpec((_BN_ROWS, d), lambda i: (i, 0)),
            pl.BlockSpec((g, d), lambda i: (0, 0)),
        ],
        out_shape=[
            jax.ShapeDtypeStruct((n, d), jnp.float32),
            jax.ShapeDtypeStruct((g, d), jnp.float32),
        ],
    )(agg, xcur, w1f, b1f, w2f, b2f, pmat_t)


def _pool_tc_kernel(xcur, pmat_t):
    n, d = xcur.shape
    g = pmat_t.shape[1]
    grid = n // _BN_ROWS

    def body(x_ref, p_ref, pool_ref):
        @pl.when(pl.program_id(0) == 0)
        def _():
            pool_ref[...] = jnp.zeros_like(pool_ref)
        pool_ref[...] += _pool_dot(p_ref[...], x_ref[...])

    return pl.pallas_call(
        body,
        grid=(grid,),
        in_specs=[
            pl.BlockSpec((_BN_ROWS, d), lambda i: (i, 0)),
            pl.BlockSpec((_BN_ROWS, g), lambda i: (i, 0)),
        ],
        out_specs=pl.BlockSpec((g, d), lambda i: (0, 0)),
        out_shape=jax.ShapeDtypeStruct((g, d), jnp.float32),
    )(xcur, pmat_t)


def _final_tc_kernel(pooled_cat, wcat, bsum):
    g, dk = pooled_cat.shape
    dout = wcat.shape[1]

    def body(p_ref, w_ref, b_ref, o_ref):
        o_ref[...] = jnp.dot(p_ref[...], w_ref[...],
                             preferred_element_type=jnp.float32) + b_ref[...]

    return pl.pallas_call(
        body,
        out_shape=jax.ShapeDtypeStruct((g, dout), jnp.float32),
    )(pooled_cat, wcat, bsum)


def kernel(x, edge_attr, params, edge_index, batch):
    n, d = x.shape
    e = edge_index.shape[1]
    g = _G

    src = edge_index[0]
    dst = edge_index[1]
    perm = jnp.argsort(dst)
    dsts = dst[perm]
    srcs = src[perm]

    node_starts = jnp.concatenate(
        [jnp.arange(_NT, dtype=jnp.int32) * _RPT,
         jnp.array([n], dtype=jnp.int32)])
    bounds = jnp.searchsorted(dsts, node_starts, side="left").astype(jnp.int32)
    bounds = jnp.concatenate(
        [bounds, jnp.full((48 - _NT - 1,), e, dtype=jnp.int32)])

    pad = _EPAD - e
    srcs_p = jnp.concatenate([srcs, jnp.zeros((pad,), jnp.int32)])
    dsts_p = jnp.concatenate([dsts, jnp.zeros((pad,), jnp.int32)])
    perm_p = jnp.concatenate([perm.astype(jnp.int32),
                              jnp.zeros((pad,), jnp.int32)])

    onehot = (batch[:, None] == jnp.arange(g, dtype=batch.dtype)[None, :])
    onehot = onehot.astype(jnp.float32)
    counts = jnp.clip(jnp.sum(onehot, axis=0, keepdims=True), 1.0)
    pmat_t = onehot / counts

    inv = 1.0 / jnp.sqrt(1.0 + _BN_EPS)
    folded = []
    for l in range(5):
        s1 = params[f'conv{l}_bn_g'] * inv
        w1f = params[f'conv{l}_W1'] * s1[None, :]
        b1f = (params[f'conv{l}_b1'] * s1 + params[f'conv{l}_bn_b'])
        s2 = params[f'bn{l}_g'] * inv
        w2f = params[f'conv{l}_W2'] * s2[None, :]
        b2f = (params[f'conv{l}_b2'] * s2 + params[f'bn{l}_b'])
        folded.append((w1f, b1f.reshape(1, -1), w2f, b2f.reshape(1, -1)))

    wcat = jnp.concatenate([params[f'fc{i}_W'] for i in range(6)], axis=0)
    bsum = sum(params[f'fc{i}_b'] for i in range(6)).reshape(1, -1)

    h = x
    pooled_list = [_pool_tc_kernel(x, pmat_t)]
    for l in range(5):
        aggflat = _sc_message_kernel(h, edge_attr, srcs_p, dsts_p, perm_p,
                                     bounds)
        agg = aggflat.reshape(n, d)
        w1f, b1f, w2f, b2f = folded[l]
        h, pooled = _layer_tc_kernel(agg, h, w1f, b1f, w2f, b2f, pmat_t)
        pooled_list.append(pooled)

    pooled_cat = jnp.concatenate(pooled_list, axis=1)
    return _final_tc_kernel(pooled_cat, wcat, bsum)


# speedup vs baseline: 2.9985x; 1.3842x over previous
import functools

import jax
import jax.numpy as jnp
from jax import lax
from jax.experimental import pallas as pl
from jax.experimental.pallas import tpu as pltpu
from jax.experimental.pallas import tpu_sc as plsc

_N = 10000
_E = 160000
_D = 256
_G = 64
_BN_EPS = 1e-5

_NT = 32
_RPT = 312
_LAST_EXTRA = 16
_BUF_ROWS = _RPT + _LAST_EXTRA + 1
_DUMMY = _RPT + _LAST_EXTRA
_C = 40
_EPAD = _E + 2 * _C


def _read_scalar_static(vref, idx, nchunks):
    lanes = lax.iota(jnp.int32, 16)
    acc = jnp.int32(0)
    for c in range(nchunks):
        v = vref[pl.ds(c * 16, 16)]
        acc = acc + jnp.sum(jnp.where(lanes + (c * 16) == idx, v, 0))
    return acc


def _read_scalar_dyn(vref, idx):
    lanes = lax.iota(jnp.int32, 16)
    base = (idx // 16) * 16
    v = vref[pl.ds(base, 16)]
    return jnp.sum(jnp.where(lanes == idx - base, v, 0))


def _sc_message_kernel(x, ea, srcs, dsts, perm, bounds):
    n, d = x.shape
    mesh = plsc.VectorSubcoreMesh(core_axis_name="c", subcore_axis_name="s")
    nj = d // 16

    @functools.partial(
        pl.kernel,
        out_type=jax.ShapeDtypeStruct((n * d,), jnp.float32),
        mesh=mesh,
        compiler_params=pltpu.CompilerParams(needs_layout_passes=False),
        scratch_types=[
            pltpu.VMEM((_BUF_ROWS * _D,), jnp.float32),
            [pltpu.VMEM((_C, _D), jnp.float32)] * 2,
            [pltpu.VMEM((_C, _D), jnp.float32)] * 2,
            [pltpu.VMEM((_C,), jnp.int32)] * 2,
            [pltpu.VMEM((48,), jnp.int32)] * 2,
            [pltpu.VMEM((_C,), jnp.int32)] * 2,
            pltpu.VMEM((48,), jnp.int32),
            [pltpu.SemaphoreType.DMA] * 2,
            [pltpu.SemaphoreType.DMA] * 2,
        ],
    )
    def k(x_hbm, ea_hbm, src_hbm, dst_hbm, perm_hbm, bnd_hbm, out_hbm,
          aggbuf, xbuf, eabuf, sidx, didx, pidx, bbuf, sem1, sem2):
        nc = 2
        wid = lax.axis_index("s") * nc + lax.axis_index("c")
        nbase = wid * _RPT
        lanes = lax.iota(jnp.int32, 16)

        @plsc.parallel_loop(0, _BUF_ROWS * nj, 1, unroll=8)
        def zero_body(i):
            aggbuf[pl.ds(i * 16, 16)] = jnp.zeros((16,), jnp.float32)

        pltpu.sync_copy(bnd_hbm, bbuf)
        b_lo = _read_scalar_static(bbuf, wid, 3)
        b_hi = _read_scalar_static(bbuf, wid + 1, 3)
        lo_al = (b_lo // 8) * 8
        nch = (b_hi - lo_al + (_C - 1)) // _C
        npair = (nch + 1) // 2
        b_lo_v = jnp.full((16,), 0, jnp.int32) + b_lo
        b_hi_v = jnp.full((16,), 0, jnp.int32) + b_hi

        def issue(i, b):
            k0 = lo_al + i * _C
            pltpu.sync_copy(src_hbm.at[pl.ds(k0, _C)], sidx[b])
            pltpu.sync_copy(dst_hbm.at[pl.ds(k0, _C)],
                            didx[b].at[pl.ds(0, _C)])
            pltpu.sync_copy(perm_hbm.at[pl.ds(k0, _C)], pidx[b])
            pltpu.async_copy(x_hbm.at[sidx[b]], xbuf[b], sem1[b])
            pltpu.async_copy(ea_hbm.at[pidx[b]], eabuf[b], sem2[b])

        def process(i, b):
            pltpu.make_async_copy(x_hbm.at[sidx[b]], xbuf[b], sem1[b]).wait()
            pltpu.make_async_copy(ea_hbm.at[pidx[b]], eabuf[b],
                                  sem2[b]).wait()
            k0 = lo_al + i * _C

            @plsc.parallel_loop(0, _C, 1, unroll=4)
            def edge_body(e):
                ge_v = (jnp.full((16,), 0, jnp.int32) + k0) + e
                e_v = jnp.full((16,), 0, jnp.int32) + e
                drow_v = plsc.load_gather(didx[b], [e_v])
                valid_v = (ge_v >= b_lo_v) & (ge_v < b_hi_v)
                rowloc_v = jnp.where(valid_v, drow_v - nbase, _DUMMY)
                addr0_v = rowloc_v * _D + lanes
                for j in range(nj):
                    xv = xbuf[b][e, pl.ds(j * 16, 16)]
                    ev = eabuf[b][e, pl.ds(j * 16, 16)]
                    m = jnp.maximum(xv + ev, 0.0)
                    plsc.addupdate_scatter(aggbuf, [addr0_v + j * 16], m)

        @pl.when(nch > 0)
        def _():
            issue(0, 0)

            def pair_body(p, _):
                i0 = 2 * p

                @pl.when(i0 + 1 < nch)
                def _():
                    issue(i0 + 1, 1)
                process(i0, 0)

                @pl.when(i0 + 2 < nch)
                def _():
                    issue(i0 + 2, 0)

                @pl.when(i0 + 1 < nch)
                def _():
                    process(i0 + 1, 1)
                return 0
            lax.fori_loop(0, npair, pair_body, 0)

        pltpu.sync_copy(aggbuf.at[pl.ds(0, _RPT * _D)],
                        out_hbm.at[pl.ds(nbase * _D, _RPT * _D)])

        @pl.when(wid == _NT - 1)
        def _():
            pltpu.sync_copy(aggbuf.at[pl.ds(_RPT * _D, _LAST_EXTRA * _D)],
                            out_hbm.at[pl.ds((_NT * _RPT) * _D,
                                             _LAST_EXTRA * _D)])

    return k(x, ea, srcs, dsts, perm, bounds)


_BN_ROWS = 2000


def _pool_dot(pt_block, h_block):
    return lax.dot_general(pt_block, h_block, (((0,), (0,)), ((), ())),
                           preferred_element_type=jnp.float32)


def _layer_tc_kernel(agg, xcur, w1f, b1f, w2f, b2f, pmat_t):
    n, d = xcur.shape
    g = pmat_t.shape[1]
    grid = n // _BN_ROWS

    def body(agg_ref, x_ref, w1_ref, b1_ref, w2_ref, b2_ref, p_ref,
             h_ref, pool_ref):
        z = agg_ref[...] + x_ref[...]
        t = jnp.dot(z, w1_ref[...], preferred_element_type=jnp.float32)
        t = jnp.maximum(t + b1_ref[...], 0.0)
        h = jnp.dot(t, w2_ref[...], preferred_element_type=jnp.float32)
        h = jnp.maximum(h + b2_ref[...], 0.0)
        h_ref[...] = h

        @pl.when(pl.program_id(0) == 0)
        def _():
            pool_ref[...] = jnp.zeros_like(pool_ref)
        pool_ref[...] += _pool_dot(p_ref[...], h)

    return pl.pallas_call(
        body,
        grid=(grid,),
        in_specs=[
            pl.BlockSpec((_BN_ROWS, d), lambda i: (i, 0)),
            pl.BlockSpec((_BN_ROWS, d), lambda i: (i, 0)),
            pl.BlockSpec((d, d), lambda i: (0, 0)),
            pl.BlockSpec((1, d), lambda i: (0, 0)),
            pl.BlockSpec((d, d), lambda i: (0, 0)),
            pl.BlockSpec((1, d), lambda i: (0, 0)),
            pl.BlockSpec((_BN_ROWS, g), lambda i: (i, 0)),
        ],
        out_specs=[
            pl.BlockSpec((_BN_ROWS, d), lambda i: (i, 0)),
            pl.BlockSpec((g, d), lambda i: (0, 0)),
        ],
        out_shape=[
            jax.ShapeDtypeStruct((n, d), jnp.float32),
            jax.ShapeDtypeStruct((g, d), jnp.float32),
        ],
    )(agg, xcur, w1f, b1f, w2f, b2f, pmat_t)


def _pool_tc_kernel(xcur, pmat_t):
    n, d = xcur.shape
    g = pmat_t.shape[1]
    grid = n // _BN_ROWS

    def body(x_ref, p_ref, pool_ref):
        @pl.when(pl.program_id(0) == 0)
        def _():
            pool_ref[...] = jnp.zeros_like(pool_ref)
        pool_ref[...] += _pool_dot(p_ref[...], x_ref[...])

    return pl.pallas_call(
        body,
        grid=(grid,),
        in_specs=[
            pl.BlockSpec((_BN_ROWS, d), lambda i: (i, 0)),
            pl.BlockSpec((_BN_ROWS, g), lambda i: (i, 0)),
        ],
        out_specs=pl.BlockSpec((g, d), lambda i: (0, 0)),
        out_shape=jax.ShapeDtypeStruct((g, d), jnp.float32),
    )(xcur, pmat_t)


def _final_tc_kernel(pooled_cat, wcat, bsum):
    g, dk = pooled_cat.shape
    dout = wcat.shape[1]

    def body(p_ref, w_ref, b_ref, o_ref):
        o_ref[...] = jnp.dot(p_ref[...], w_ref[...],
                             preferred_element_type=jnp.float32) + b_ref[...]

    return pl.pallas_call(
        body,
        out_shape=jax.ShapeDtypeStruct((g, dout), jnp.float32),
    )(pooled_cat, wcat, bsum)


def kernel(x, edge_attr, params, edge_index, batch):
    n, d = x.shape
    e = edge_index.shape[1]
    g = _G

    src = edge_index[0]
    dst = edge_index[1]
    perm = jnp.argsort(dst)
    dsts = dst[perm]
    srcs = src[perm]

    node_starts = jnp.concatenate(
        [jnp.arange(_NT, dtype=jnp.int32) * _RPT,
         jnp.array([n], dtype=jnp.int32)])
    bounds = jnp.searchsorted(dsts, node_starts, side="left").astype(jnp.int32)
    bounds = jnp.concatenate(
        [bounds, jnp.full((48 - _NT - 1,), e, dtype=jnp.int32)])

    pad = _EPAD - e
    srcs_p = jnp.concatenate([srcs, jnp.zeros((pad,), jnp.int32)])
    dsts_p = jnp.concatenate([dsts, jnp.zeros((pad,), jnp.int32)])
    perm_p = jnp.concatenate([perm.astype(jnp.int32),
                              jnp.zeros((pad,), jnp.int32)])

    onehot = (batch[:, None] == jnp.arange(g, dtype=batch.dtype)[None, :])
    onehot = onehot.astype(jnp.float32)
    counts = jnp.clip(jnp.sum(onehot, axis=0, keepdims=True), 1.0)
    pmat_t = onehot / counts

    inv = 1.0 / jnp.sqrt(1.0 + _BN_EPS)
    folded = []
    for l in range(5):
        s1 = params[f'conv{l}_bn_g'] * inv
        w1f = params[f'conv{l}_W1'] * s1[None, :]
        b1f = (params[f'conv{l}_b1'] * s1 + params[f'conv{l}_bn_b'])
        s2 = params[f'bn{l}_g'] * inv
        w2f = params[f'conv{l}_W2'] * s2[None, :]
        b2f = (params[f'conv{l}_b2'] * s2 + params[f'bn{l}_b'])
        folded.append((w1f, b1f.reshape(1, -1), w2f, b2f.reshape(1, -1)))

    wcat = jnp.concatenate([params[f'fc{i}_W'] for i in range(6)], axis=0)
    bsum = sum(params[f'fc{i}_b'] for i in range(6)).reshape(1, -1)

    h = x
    pooled_list = [_pool_tc_kernel(x, pmat_t)]
    for l in range(5):
        aggflat = _sc_message_kernel(h, edge_attr, srcs_p, dsts_p, perm_p,
                                     bounds)
        agg = aggflat.reshape(n, d)
        w1f, b1f, w2f, b2f = folded[l]
        h, pooled = _layer_tc_kernel(agg, h, w1f, b1f, w2f, b2f, pmat_t)
        pooled_list.append(pooled)

    pooled_cat = jnp.concatenate(pooled_list, axis=1)
    return _final_tc_kernel(pooled_cat, wcat, bsum)
